# baseline (speedup 1.0000x reference)
import functools

import jax
import jax.numpy as jnp
from jax import lax
from jax.experimental import pallas as pl
from jax.experimental.pallas import tpu as pltpu
from jax.experimental.pallas import tpu_sc as plsc

_LANES = 16
_CHUNK = 128
_GROUP = 5
_NC = 2
_NS = 16
_NW = _NC * _NS
_NPAD = 102400
_ZROWS = 160
_EPS = 1e-5


@functools.lru_cache(maxsize=None)
def _make_sc_agg(nslab, n_nodes, n_rows):
    rows_per_tile = n_rows // _NW
    groups = rows_per_tile // _GROUP
    npt = _NPAD // _NS
    nzcopy = npt // _ZROWS

    mesh = plsc.VectorSubcoreMesh(
        core_axis_name="c", subcore_axis_name="s",
        num_cores=_NC, num_subcores=_NS)

    gb = _GROUP * _CHUNK

    scratch = [
        pltpu.VMEM((2, _GROUP, _CHUNK), jnp.int32),
        pltpu.VMEM((2, _GROUP, _CHUNK), jnp.int32),
        pltpu.VMEM((gb, _LANES), jnp.float32),
        pltpu.VMEM((gb, _LANES), jnp.float32),
        pltpu.VMEM((_ZROWS, _LANES), jnp.float32),
        pltpu.VMEM_SHARED((_NPAD, _LANES), jnp.float32),
        pltpu.SemaphoreType.DMA,
        pltpu.SemaphoreType.DMA,
        pltpu.SemaphoreType.DMA,
        pltpu.SemaphoreType.DMA,
    ]

    def body(*refs):
        slabs = refs[:nslab]
        sd_hbm, out_hbm = refs[nslab:nslab + 2]
        sdb0, sdb1, rows0, rows1, zbuf, acc, sg0, sg1, ss0, ss1 = \
            refs[nslab + 2:]
        c = lax.axis_index("c")
        s = lax.axis_index("s")
        wid = c * _NS + s
        rowbase = wid * rows_per_tile
        zrow = s * npt
        bufs = ((sdb0, rows0, sg0, ss0), (sdb1, rows1, sg1, ss1))

        def load_sd(g, sdb):
            pltpu.sync_copy(sd_hbm.at[:, pl.ds(rowbase + g * _GROUP, _GROUP), :],
                            sdb)

        def fire_gathers(slab_ref, sdb, rowsb, sem):
            for j in range(_GROUP):
                pltpu.async_copy(slab_ref.at[sdb.at[0, j]],
                                 rowsb.at[pl.ds(j * _CHUNK, _CHUNK), :], sem)

        def fire_scatters(sdb, rowsb, sem):
            for j in range(_GROUP):
                pltpu.async_copy(rowsb.at[pl.ds(j * _CHUNK, _CHUNK), :],
                                 acc.at[sdb.at[1, j]], sem, add=True)

        def drain(sem, rowsb):
            pltpu.make_async_copy(out_hbm.at[0, pl.ds(0, gb), :], rowsb,
                                  sem).wait()

        @pl.loop(0, _ZROWS)
        def _(i):
            zbuf[i, :] = jnp.zeros((_LANES,), jnp.float32)

        for slab in range(nslab):
            for z in range(nzcopy):
                pltpu.sync_copy(zbuf, acc.at[pl.ds(zrow + z * _ZROWS, _ZROWS), :])
            plsc.subcore_barrier()

            slab_ref = slabs[slab]
            for b in range(2):
                sdb, rowsb, sg, _ = bufs[b]
                load_sd(b, sdb)
                fire_gathers(slab_ref, sdb, rowsb, sg)

            @pl.loop(0, groups, step=2)
            def _(gg):
                for b in range(2):
                    sdb, rowsb, sg, ss = bufs[b]
                    drain(sg, rowsb)
                    fire_scatters(sdb, rowsb, ss)
                for b in range(2):
                    sdb, rowsb, sg, ss = bufs[b]
                    g = gg + b
                    drain(ss, rowsb)

                    @pl.when(g + 2 < groups)
                    def _():
                        load_sd(g + 2, sdb)
                        fire_gathers(slab_ref, sdb, rowsb, sg)
            plsc.subcore_barrier()

            pltpu.sync_copy(acc.at[pl.ds(zrow, npt), :],
                            out_hbm.at[slab * _NC + c, pl.ds(zrow, npt), :])

    out_type = jax.ShapeDtypeStruct((nslab * _NC, _NPAD, _LANES),
                                    jnp.float32)
    return pl.kernel(body, out_type=out_type, mesh=mesh,
                     scratch_types=scratch,
                     compiler_params=pltpu.CompilerParams(
                         use_tc_tiling_on_sc=False))


_R = 128


@functools.lru_cache(maxsize=None)
def _make_pre(nslab_in, d_out, n_nodes):
    dp = 8 * d_out
    npk = _NPAD // 8
    nblk = npk // _R
    nreal = n_nodes // 8

    def body(*refs):
        agg_ref = refs[0]
        hs = refs[1:1 + nslab_in]
        wrel, wroot, bias = refs[1 + nslab_in:4 + nslab_in]
        pre_ref, stats_ref, acc = refs[4 + nslab_in:]
        i = pl.program_id(0)
        o = bias[...]
        for s in range(nslab_in):
            aggs = agg_ref[2 * s] + agg_ref[2 * s + 1]
            o = o + jnp.dot(aggs, wrel[s], preferred_element_type=jnp.float32)
            o = o + jnp.dot(hs[s][...], wroot[s],
                            preferred_element_type=jnp.float32)
        pre_ref[...] = o
        rid = lax.broadcasted_iota(jnp.int32, (_R, 1), 0) + i * _R
        valid = (rid < nreal).astype(jnp.float32)
        ov = o * valid
        s1f = jnp.sum(ov, axis=0, keepdims=True)
        s2f = jnp.sum(ov * o, axis=0, keepdims=True)
        s1 = s1f[:, 0:d_out]
        s2 = s2f[:, 0:d_out]
        for k in range(1, 8):
            s1 = s1 + s1f[:, k * d_out:(k + 1) * d_out]
            s2 = s2 + s2f[:, k * d_out:(k + 1) * d_out]
        st = jnp.concatenate([s1, s2], axis=0)

        @pl.when(i == 0)
        def _():
            acc[...] = st

        @pl.when(i > 0)
        def _():
            acc[...] += st

        @pl.when(i == nblk - 1)
        def _():
            stats_ref[...] = acc[...]

    in_specs = (
        [pl.BlockSpec((nslab_in * 2, _R, 8 * _LANES), lambda i: (0, i, 0))]
        + [pl.BlockSpec((_R, 8 * _LANES), lambda i: (i, 0))] * nslab_in
        + [pl.BlockSpec((nslab_in, 8 * _LANES, dp), lambda i: (0, 0, 0)),
           pl.BlockSpec((nslab_in, 8 * _LANES, dp), lambda i: (0, 0, 0)),
           pl.BlockSpec((1, dp), lambda i: (0, 0))]
    )
    out_specs = [pl.BlockSpec((_R, dp), lambda i: (i, 0)),
                 pl.BlockSpec((2, d_out), lambda i: (0, 0))]
    out_shape = [jax.ShapeDtypeStruct((npk, dp), jnp.float32),
                 jax.ShapeDtypeStruct((2, d_out), jnp.float32)]

    return pl.pallas_call(
        body, grid=(nblk,), in_specs=in_specs, out_specs=out_specs,
        out_shape=out_shape,
        scratch_shapes=[pltpu.VMEM((2, d_out), jnp.float32)])


@functools.lru_cache(maxsize=None)
def _make_norm(d, n_nodes):
    dp = 8 * d
    nslab_out = d // _LANES
    npk = _NPAD // 8
    nblk = npk // _R

    def body(pre_ref, stats_ref, bng, bnb, gng, gnb, gna, *outs):
        n = jnp.float32(n_nodes)
        m = stats_ref[0:1, :] / n
        v = stats_ref[1:2, :] / n - m * m
        bg = bng[...]
        bb = bnb[...]
        gg = gng[...]
        gb = gnb[...]
        ga = gna[...]
        inv_s = lax.rsqrt(v + _EPS)
        cc = bb * (1.0 - ga)
        v2 = bg * bg * v * inv_s * inv_s + cc * cc
        t_inv = lax.rsqrt(v2 + _EPS)
        a_coef = gg * bg * inv_s * t_inv
        b_coef = gg * cc * t_inv + gb - a_coef * m
        a_t = jnp.tile(a_coef, (1, 8))
        b_t = jnp.tile(b_coef, (1, 8))
        h = jnp.maximum(pre_ref[...] * a_t + b_t, 0.0)
        for s in range(nslab_out):
            if nslab_out == 1:
                outs[s][...] = h
            else:
                outs[s][...] = jnp.concatenate(
                    [h[:, k * d + s * _LANES:k * d + (s + 1) * _LANES]
                     for k in range(8)], axis=1)

    in_specs = (
        [pl.BlockSpec((_R, dp), lambda i: (i, 0)),
         pl.BlockSpec((2, d), lambda i: (0, 0))]
        + [pl.BlockSpec((1, d), lambda i: (0, 0))] * 5
    )
    out_specs = [pl.BlockSpec((_R, 8 * _LANES), lambda i: (i, 0))] * nslab_out
    out_shape = [jax.ShapeDtypeStruct((npk, 8 * _LANES), jnp.float32)] * nslab_out

    return pl.pallas_call(
        body, grid=(nblk,), in_specs=in_specs, out_specs=out_specs,
        out_shape=out_shape)


@functools.lru_cache(maxsize=None)
def _make_heads(nslab_in, d_out, n_nodes):
    dp = 8 * d_out
    npk = _NPAD // 8
    nblk = npk // _R

    def body(*refs):
        agg_ref = refs[0]
        hs = refs[1:1 + nslab_in]
        wrel_mu, wroot_mu, b_mu, wrel_lv, wroot_lv, b_lv = \
            refs[1 + nslab_in:7 + nslab_in]
        mu_ref, lv_ref = refs[7 + nslab_in:]
        mu = b_mu[...]
        lv = b_lv[...]
        for s in range(nslab_in):
            aggs = agg_ref[2 * s] + agg_ref[2 * s + 1]
            hss = hs[s][...]
            mu = mu + jnp.dot(aggs, wrel_mu[s], preferred_element_type=jnp.float32)
            mu = mu + jnp.dot(hss, wroot_mu[s], preferred_element_type=jnp.float32)
            lv = lv + jnp.dot(aggs, wrel_lv[s], preferred_element_type=jnp.float32)
            lv = lv + jnp.dot(hss, wroot_lv[s], preferred_element_type=jnp.float32)
        mu_ref[...] = mu
        lv_ref[...] = lv

    wspec = pl.BlockSpec((nslab_in, 8 * _LANES, dp), lambda i: (0, 0, 0))
    bspec = pl.BlockSpec((1, dp), lambda i: (0, 0))
    in_specs = (
        [pl.BlockSpec((nslab_in * 2, _R, 8 * _LANES), lambda i: (0, i, 0))]
        + [pl.BlockSpec((_R, 8 * _LANES), lambda i: (i, 0))] * nslab_in
        + [wspec, wspec, bspec, wspec, wspec, bspec]
    )
    out_specs = [pl.BlockSpec((_R, dp), lambda i: (i, 0))] * 2
    out_shape = [jax.ShapeDtypeStruct((npk, dp), jnp.float32)] * 2

    return pl.pallas_call(
        body, grid=(nblk,), in_specs=in_specs, out_specs=out_specs,
        out_shape=out_shape)


def _wpack(w, nslab):
    eye = jnp.eye(8, dtype=w.dtype)
    return jnp.stack([jnp.kron(eye, w[16 * s:16 * (s + 1), :])
                      for s in range(nslab)])


def kernel(x, edge_index,
           conv1_w_rel, conv1_w_root, conv1_b,
           conv2_w_rel, conv2_w_root, conv2_b,
           conv3_w_rel, conv3_w_root, conv3_b,
           conv_mu_w_rel, conv_mu_w_root, conv_mu_b,
           conv_logvar_w_rel, conv_logvar_w_root, conv_logvar_b,
           bn1_g, bn1_b, bn2_g, bn2_b, bn3_g, bn3_b,
           gn1_g, gn1_b, gn1_a, gn2_g, gn2_b, gn2_a,
           gn3_g, gn3_b, gn3_a):
    n = x.shape[0]
    e = edge_index.shape[1]
    e_pad = _NW * 400 * _CHUNK
    fill_dst = n + jax.lax.rem(jnp.arange(e_pad - e, dtype=jnp.int32),
                               jnp.int32(_NPAD - n))
    fill = jnp.stack([jnp.zeros(e_pad - e, jnp.int32), fill_dst])
    n_rows = e_pad // _CHUNK
    sd = jnp.concatenate([edge_index, fill], axis=1).reshape(
        2, n_rows, _CHUNK)

    row2 = lambda a: a.reshape(1, -1)
    tile8 = lambda a: jnp.tile(a, 8).reshape(1, -1)
    unpk = lambda a: a.reshape(_NPAD, _LANES)
    pk = lambda a: a.reshape(a.shape[0], _NPAD // 8, 8 * _LANES)

    x16 = jnp.pad(x, ((0, _NPAD - n), (0, _LANES - 1)))
    x16p = x16.reshape(_NPAD // 8, 8 * _LANES)
    w1r = jnp.pad(conv1_w_rel, ((0, _LANES - 1), (0, 0)))
    w1t = jnp.pad(conv1_w_root, ((0, _LANES - 1), (0, 0)))
    agg1 = pk(_make_sc_agg(1, n, n_rows)(x16, sd))
    pre1, st1 = _make_pre(1, 16, n)(agg1, x16p, _wpack(w1r, 1),
                                    _wpack(w1t, 1), tile8(conv1_b))
    (h1,) = _make_norm(16, n)(pre1, st1, row2(bn1_g), row2(bn1_b),
                              row2(gn1_g), row2(gn1_b), row2(gn1_a))

    agg2 = pk(_make_sc_agg(1, n, n_rows)(unpk(h1), sd))
    pre2, st2 = _make_pre(1, 32, n)(agg2, h1, _wpack(conv2_w_rel, 1),
                                    _wpack(conv2_w_root, 1), tile8(conv2_b))
    h2a, h2b = _make_norm(32, n)(pre2, st2, row2(bn2_g), row2(bn2_b),
                                 row2(gn2_g), row2(gn2_b), row2(gn2_a))

    agg3 = pk(_make_sc_agg(2, n, n_rows)(unpk(h2a), unpk(h2b), sd))
    pre3, st3 = _make_pre(2, 64, n)(agg3, h2a, h2b, _wpack(conv3_w_rel, 2),
                                    _wpack(conv3_w_root, 2), tile8(conv3_b))
    h3 = _make_norm(64, n)(pre3, st3, row2(bn3_g), row2(bn3_b),
                           row2(gn3_g), row2(gn3_b), row2(gn3_a))

    agg4 = pk(_make_sc_agg(4, n, n_rows)(unpk(h3[0]), unpk(h3[1]),
                                         unpk(h3[2]), unpk(h3[3]), sd))
    mup, lvp = _make_heads(4, 64, n)(
        agg4, h3[0], h3[1], h3[2], h3[3],
        _wpack(conv_mu_w_rel, 4), _wpack(conv_mu_w_root, 4),
        tile8(conv_mu_b),
        _wpack(conv_logvar_w_rel, 4), _wpack(conv_logvar_w_root, 4),
        tile8(conv_logvar_b))
    return (mup[:n // 8].reshape(n, 64),
            lvp[:n // 8].reshape(n, 64))


# speedup vs baseline: 2.0179x; 2.0179x over previous
import functools

import jax
import jax.numpy as jnp
from jax import lax
from jax.experimental import pallas as pl
from jax.experimental.pallas import tpu as pltpu
from jax.experimental.pallas import tpu_sc as plsc

_LANES = 16
_CHUNK = 128
_GROUP = 5
_NC = 2
_NS = 16
_NW = _NC * _NS
_NPAD = 102400
_ZROWS = 160
_EPS = 1e-5


@functools.lru_cache(maxsize=None)
def _make_sc_agg(nslab, n_nodes, n_rows):
    rows_per_tile = n_rows // _NW
    groups = rows_per_tile // _GROUP
    npt = _NPAD // _NS
    nzcopy = npt // _ZROWS

    mesh = plsc.VectorSubcoreMesh(
        core_axis_name="c", subcore_axis_name="s",
        num_cores=_NC, num_subcores=_NS)

    gb = _GROUP * _CHUNK

    scratch = [
        pltpu.VMEM((2, _GROUP, _CHUNK), jnp.int32),
        pltpu.VMEM((2, _GROUP, _CHUNK), jnp.int32),
        pltpu.VMEM((gb, _LANES), jnp.float32),
        pltpu.VMEM((gb, _LANES), jnp.float32),
        pltpu.VMEM((_ZROWS, _LANES), jnp.float32),
        pltpu.VMEM_SHARED((_NPAD, _LANES), jnp.float32),
        pltpu.SemaphoreType.DMA,
        pltpu.SemaphoreType.DMA,
        pltpu.SemaphoreType.DMA,
        pltpu.SemaphoreType.DMA,
    ]

    def body(*refs):
        slabs = refs[:nslab]
        sd_hbm, out_hbm = refs[nslab:nslab + 2]
        sdb0, sdb1, rows0, rows1, zbuf, acc, sg0, sg1, ss0, ss1 = \
            refs[nslab + 2:]
        c = lax.axis_index("c")
        s = lax.axis_index("s")
        wid = c * _NS + s
        rowbase = wid * rows_per_tile
        zrow = s * npt
        bufs = ((sdb0, rows0, sg0, ss0), (sdb1, rows1, sg1, ss1))

        def load_sd(g, sdb):
            pltpu.sync_copy(sd_hbm.at[:, pl.ds(rowbase + g * _GROUP, _GROUP), :],
                            sdb)

        def fire_gathers(slab_ref, sdb, rowsb, sem):
            for j in range(_GROUP):
                pltpu.async_copy(slab_ref.at[sdb.at[0, j]],
                                 rowsb.at[pl.ds(j * _CHUNK, _CHUNK), :], sem)

        def fire_scatters(sdb, rowsb, sem):
            for j in range(_GROUP):
                pltpu.async_copy(rowsb.at[pl.ds(j * _CHUNK, _CHUNK), :],
                                 acc.at[sdb.at[1, j]], sem, add=True)

        def drain(sem, rowsb):
            pltpu.make_async_copy(out_hbm.at[0, pl.ds(0, gb), :], rowsb,
                                  sem).wait()

        @pl.loop(0, _ZROWS)
        def _(i):
            zbuf[i, :] = jnp.zeros((_LANES,), jnp.float32)

        for slab in range(nslab):
            for z in range(nzcopy):
                pltpu.sync_copy(zbuf, acc.at[pl.ds(zrow + z * _ZROWS, _ZROWS), :])
            plsc.subcore_barrier()

            slab_ref = slabs[slab]
            for b in range(2):
                sdb, rowsb, sg, _ = bufs[b]
                load_sd(b, sdb)
                fire_gathers(slab_ref, sdb, rowsb, sg)

            @pl.loop(0, groups, step=2)
            def _(gg):
                for b in range(2):
                    sdb, rowsb, sg, ss = bufs[b]
                    drain(sg, rowsb)
                    fire_scatters(sdb, rowsb, ss)
                for b in range(2):
                    sdb, rowsb, sg, ss = bufs[b]
                    g = gg + b
                    drain(ss, rowsb)

                    @pl.when(g + 2 < groups)
                    def _():
                        load_sd(g + 2, sdb)
                        fire_gathers(slab_ref, sdb, rowsb, sg)
            plsc.subcore_barrier()

            pltpu.sync_copy(acc.at[pl.ds(zrow, npt), :],
                            out_hbm.at[slab * _NC + c, pl.ds(zrow, npt), :])

    out_type = jax.ShapeDtypeStruct((nslab * _NC, _NPAD, _LANES),
                                    jnp.float32)
    return pl.kernel(body, out_type=out_type, mesh=mesh,
                     scratch_types=scratch,
                     compiler_params=pltpu.CompilerParams(
                         use_tc_tiling_on_sc=False))


_R = 128


@functools.lru_cache(maxsize=None)
def _make_pre(nslab_in, d_out, n_nodes):
    dp = 8 * d_out
    npk = _NPAD // 8
    nblk = npk // _R
    nreal = n_nodes // 8

    def body(*refs):
        agg_ref = refs[0]
        hs = refs[1:1 + nslab_in]
        wrel, wroot, bias = refs[1 + nslab_in:4 + nslab_in]
        pre_ref, stats_ref, acc = refs[4 + nslab_in:]
        i = pl.program_id(0)
        o = bias[...]
        for s in range(nslab_in):
            aggs = agg_ref[2 * s] + agg_ref[2 * s + 1]
            o = o + jnp.dot(aggs, wrel[s], preferred_element_type=jnp.float32)
            o = o + jnp.dot(hs[s][...], wroot[s],
                            preferred_element_type=jnp.float32)
        pre_ref[...] = o
        rid = lax.broadcasted_iota(jnp.int32, (_R, 1), 0) + i * _R
        valid = (rid < nreal).astype(jnp.float32)
        ov = o * valid
        s1f = jnp.sum(ov, axis=0, keepdims=True)
        s2f = jnp.sum(ov * o, axis=0, keepdims=True)
        s1 = s1f[:, 0:d_out]
        s2 = s2f[:, 0:d_out]
        for k in range(1, 8):
            s1 = s1 + s1f[:, k * d_out:(k + 1) * d_out]
            s2 = s2 + s2f[:, k * d_out:(k + 1) * d_out]
        st = jnp.concatenate([s1, s2], axis=0)

        @pl.when(i == 0)
        def _():
            acc[...] = st

        @pl.when(i > 0)
        def _():
            acc[...] += st

        @pl.when(i == nblk - 1)
        def _():
            stats_ref[...] = acc[...]

    in_specs = (
        [pl.BlockSpec((nslab_in * 2, _R, 8 * _LANES), lambda i: (0, i, 0))]
        + [pl.BlockSpec((_R, 8 * _LANES), lambda i: (i, 0))] * nslab_in
        + [pl.BlockSpec((nslab_in, 8 * _LANES, dp), lambda i: (0, 0, 0)),
           pl.BlockSpec((nslab_in, 8 * _LANES, dp), lambda i: (0, 0, 0)),
           pl.BlockSpec((1, dp), lambda i: (0, 0))]
    )
    out_specs = [pl.BlockSpec((_R, dp), lambda i: (i, 0)),
                 pl.BlockSpec((2, d_out), lambda i: (0, 0))]
    out_shape = [jax.ShapeDtypeStruct((npk, dp), jnp.float32),
                 jax.ShapeDtypeStruct((2, d_out), jnp.float32)]

    return pl.pallas_call(
        body, grid=(nblk,), in_specs=in_specs, out_specs=out_specs,
        out_shape=out_shape,
        scratch_shapes=[pltpu.VMEM((2, d_out), jnp.float32)])


@functools.lru_cache(maxsize=None)
def _make_norm(d, n_nodes):
    dp = 8 * d
    nslab_out = d // _LANES
    npk = _NPAD // 8
    nblk = npk // _R

    def body(pre_ref, stats_ref, bng, bnb, gng, gnb, gna, *outs):
        n = jnp.float32(n_nodes)
        m = stats_ref[0:1, :] / n
        v = stats_ref[1:2, :] / n - m * m
        bg = bng[...]
        bb = bnb[...]
        gg = gng[...]
        gb = gnb[...]
        ga = gna[...]
        inv_s = lax.rsqrt(v + _EPS)
        cc = bb * (1.0 - ga)
        v2 = bg * bg * v * inv_s * inv_s + cc * cc
        t_inv = lax.rsqrt(v2 + _EPS)
        a_coef = gg * bg * inv_s * t_inv
        b_coef = gg * cc * t_inv + gb - a_coef * m
        a_t = jnp.tile(a_coef, (1, 8))
        b_t = jnp.tile(b_coef, (1, 8))
        h = jnp.maximum(pre_ref[...] * a_t + b_t, 0.0)
        for s in range(nslab_out):
            if nslab_out == 1:
                outs[s][...] = h
            else:
                outs[s][...] = jnp.concatenate(
                    [h[:, k * d + s * _LANES:k * d + (s + 1) * _LANES]
                     for k in range(8)], axis=1)

    in_specs = (
        [pl.BlockSpec((_R, dp), lambda i: (i, 0)),
         pl.BlockSpec((2, d), lambda i: (0, 0))]
        + [pl.BlockSpec((1, d), lambda i: (0, 0))] * 5
    )
    out_specs = [pl.BlockSpec((_R, 8 * _LANES), lambda i: (i, 0))] * nslab_out
    out_shape = [jax.ShapeDtypeStruct((npk, 8 * _LANES), jnp.float32)] * nslab_out

    return pl.pallas_call(
        body, grid=(nblk,), in_specs=in_specs, out_specs=out_specs,
        out_shape=out_shape)


@functools.lru_cache(maxsize=None)
def _make_heads(nslab_in, d_out, n_nodes):
    dp = 8 * d_out
    npk = _NPAD // 8
    nblk = npk // _R

    def body(*refs):
        agg_ref = refs[0]
        hs = refs[1:1 + nslab_in]
        wrel_mu, wroot_mu, b_mu, wrel_lv, wroot_lv, b_lv = \
            refs[1 + nslab_in:7 + nslab_in]
        mu_ref, lv_ref = refs[7 + nslab_in:]
        mu = b_mu[...]
        lv = b_lv[...]
        for s in range(nslab_in):
            aggs = agg_ref[2 * s] + agg_ref[2 * s + 1]
            hss = hs[s][...]
            mu = mu + jnp.dot(aggs, wrel_mu[s], preferred_element_type=jnp.float32)
            mu = mu + jnp.dot(hss, wroot_mu[s], preferred_element_type=jnp.float32)
            lv = lv + jnp.dot(aggs, wrel_lv[s], preferred_element_type=jnp.float32)
            lv = lv + jnp.dot(hss, wroot_lv[s], preferred_element_type=jnp.float32)
        mu_ref[...] = mu
        lv_ref[...] = lv

    wspec = pl.BlockSpec((nslab_in, 8 * _LANES, dp), lambda i: (0, 0, 0))
    bspec = pl.BlockSpec((1, dp), lambda i: (0, 0))
    in_specs = (
        [pl.BlockSpec((nslab_in * 2, _R, 8 * _LANES), lambda i: (0, i, 0))]
        + [pl.BlockSpec((_R, 8 * _LANES), lambda i: (i, 0))] * nslab_in
        + [wspec, wspec, bspec, wspec, wspec, bspec]
    )
    out_specs = [pl.BlockSpec((_R, dp), lambda i: (i, 0))] * 2
    out_shape = [jax.ShapeDtypeStruct((npk, dp), jnp.float32)] * 2

    return pl.pallas_call(
        body, grid=(nblk,), in_specs=in_specs, out_specs=out_specs,
        out_shape=out_shape)


def _wpack(w, nslab):
    eye = jnp.eye(8, dtype=w.dtype)
    return jnp.stack([jnp.kron(eye, w[16 * s:16 * (s + 1), :])
                      for s in range(nslab)])


def kernel(x, edge_index,
           conv1_w_rel, conv1_w_root, conv1_b,
           conv2_w_rel, conv2_w_root, conv2_b,
           conv3_w_rel, conv3_w_root, conv3_b,
           conv_mu_w_rel, conv_mu_w_root, conv_mu_b,
           conv_logvar_w_rel, conv_logvar_w_root, conv_logvar_b,
           bn1_g, bn1_b, bn2_g, bn2_b, bn3_g, bn3_b,
           gn1_g, gn1_b, gn1_a, gn2_g, gn2_b, gn2_a,
           gn3_g, gn3_b, gn3_a):
    n = x.shape[0]
    e = edge_index.shape[1]
    e_pad = _NW * 400 * _CHUNK
    ar = jnp.arange(e_pad - e, dtype=jnp.int32)
    fill_src = jax.lax.rem(ar * 4099, jnp.int32(n))
    fill_dst = n + jax.lax.rem(ar, jnp.int32(_NPAD - n))
    fill = jnp.stack([fill_src, fill_dst])
    n_rows = e_pad // _CHUNK
    sd = jnp.concatenate([edge_index, fill], axis=1).reshape(
        2, n_rows, _CHUNK)

    row2 = lambda a: a.reshape(1, -1)
    tile8 = lambda a: jnp.tile(a, 8).reshape(1, -1)
    unpk = lambda a: a.reshape(_NPAD, _LANES)
    pk = lambda a: a.reshape(a.shape[0], _NPAD // 8, 8 * _LANES)

    x16 = jnp.pad(x, ((0, _NPAD - n), (0, _LANES - 1)))
    x16p = x16.reshape(_NPAD // 8, 8 * _LANES)
    w1r = jnp.pad(conv1_w_rel, ((0, _LANES - 1), (0, 0)))
    w1t = jnp.pad(conv1_w_root, ((0, _LANES - 1), (0, 0)))
    agg1 = pk(_make_sc_agg(1, n, n_rows)(x16, sd))
    pre1, st1 = _make_pre(1, 16, n)(agg1, x16p, _wpack(w1r, 1),
                                    _wpack(w1t, 1), tile8(conv1_b))
    (h1,) = _make_norm(16, n)(pre1, st1, row2(bn1_g), row2(bn1_b),
                              row2(gn1_g), row2(gn1_b), row2(gn1_a))

    agg2 = pk(_make_sc_agg(1, n, n_rows)(unpk(h1), sd))
    pre2, st2 = _make_pre(1, 32, n)(agg2, h1, _wpack(conv2_w_rel, 1),
                                    _wpack(conv2_w_root, 1), tile8(conv2_b))
    h2a, h2b = _make_norm(32, n)(pre2, st2, row2(bn2_g), row2(bn2_b),
                                 row2(gn2_g), row2(gn2_b), row2(gn2_a))

    agg3 = pk(_make_sc_agg(2, n, n_rows)(unpk(h2a), unpk(h2b), sd))
    pre3, st3 = _make_pre(2, 64, n)(agg3, h2a, h2b, _wpack(conv3_w_rel, 2),
                                    _wpack(conv3_w_root, 2), tile8(conv3_b))
    h3 = _make_norm(64, n)(pre3, st3, row2(bn3_g), row2(bn3_b),
                           row2(gn3_g), row2(gn3_b), row2(gn3_a))

    agg4 = pk(_make_sc_agg(4, n, n_rows)(unpk(h3[0]), unpk(h3[1]),
                                         unpk(h3[2]), unpk(h3[3]), sd))
    mup, lvp = _make_heads(4, 64, n)(
        agg4, h3[0], h3[1], h3[2], h3[3],
        _wpack(conv_mu_w_rel, 4), _wpack(conv_mu_w_root, 4),
        tile8(conv_mu_b),
        _wpack(conv_logvar_w_rel, 4), _wpack(conv_logvar_w_root, 4),
        tile8(conv_logvar_b))
    return (mup[:n // 8].reshape(n, 64),
            lvp[:n // 8].reshape(n, 64))


# speedup vs baseline: 2.6282x; 1.3025x over previous
import functools

import jax
import jax.numpy as jnp
from jax import lax
from jax.experimental import pallas as pl
from jax.experimental.pallas import tpu as pltpu
from jax.experimental.pallas import tpu_sc as plsc

_LANES = 16
_CHUNK = 125
_GROUP = 5
_NC = 2
_NS = 16
_NW = _NC * _NS
_NPAD = 102400
_ZROWS = 160
_EPS = 1e-5


@functools.lru_cache(maxsize=None)
def _make_sc_agg(nslab, n_nodes, n_rows):
    rows_per_tile = n_rows // _NW
    groups = rows_per_tile // _GROUP
    npt = _NPAD // _NS
    nzcopy = npt // _ZROWS

    mesh = plsc.VectorSubcoreMesh(
        core_axis_name="c", subcore_axis_name="s",
        num_cores=_NC, num_subcores=_NS)

    gb = _GROUP * _CHUNK

    scratch = [
        pltpu.VMEM((2, _GROUP, _CHUNK), jnp.int32),
        pltpu.VMEM((2, _GROUP, _CHUNK), jnp.int32),
        pltpu.VMEM((gb, _LANES), jnp.float32),
        pltpu.VMEM((gb, _LANES), jnp.float32),
        pltpu.VMEM((_ZROWS, _LANES), jnp.float32),
        pltpu.VMEM_SHARED((_NPAD, _LANES), jnp.float32),
        pltpu.SemaphoreType.DMA,
        pltpu.SemaphoreType.DMA,
        pltpu.SemaphoreType.DMA,
        pltpu.SemaphoreType.DMA,
    ]

    def body(*refs):
        slabs = refs[:nslab]
        sd_hbm, out_hbm = refs[nslab:nslab + 2]
        sdb0, sdb1, rows0, rows1, zbuf, acc, sg0, sg1, ss0, ss1 = \
            refs[nslab + 2:]
        c = lax.axis_index("c")
        s = lax.axis_index("s")
        wid = c * _NS + s
        rowbase = wid * rows_per_tile
        zrow = s * npt
        bufs = ((sdb0, rows0, sg0, ss0), (sdb1, rows1, sg1, ss1))

        def load_sd(g, sdb):
            pltpu.sync_copy(sd_hbm.at[:, pl.ds(rowbase + g * _GROUP, _GROUP), :],
                            sdb)

        def fire_gathers(slab_ref, sdb, rowsb, sem):
            for j in range(_GROUP):
                pltpu.async_copy(slab_ref.at[sdb.at[0, j]],
                                 rowsb.at[pl.ds(j * _CHUNK, _CHUNK), :], sem)

        def fire_scatters(sdb, rowsb, sem):
            for j in range(_GROUP):
                pltpu.async_copy(rowsb.at[pl.ds(j * _CHUNK, _CHUNK), :],
                                 acc.at[sdb.at[1, j]], sem, add=True)

        def drain(sem, rowsb):
            pltpu.make_async_copy(out_hbm.at[0, pl.ds(0, gb), :], rowsb,
                                  sem).wait()

        @pl.loop(0, _ZROWS)
        def _(i):
            zbuf[i, :] = jnp.zeros((_LANES,), jnp.float32)

        for slab in range(nslab):
            for z in range(nzcopy):
                pltpu.sync_copy(zbuf, acc.at[pl.ds(zrow + z * _ZROWS, _ZROWS), :])
            plsc.subcore_barrier()

            slab_ref = slabs[slab]
            for b in range(2):
                sdb, rowsb, sg, _ = bufs[b]
                load_sd(b, sdb)
                fire_gathers(slab_ref, sdb, rowsb, sg)

            @pl.loop(0, groups, step=2)
            def _(gg):
                for b in range(2):
                    sdb, rowsb, sg, ss = bufs[b]
                    g = gg + b
                    drain(sg, rowsb)
                    fire_scatters(sdb, rowsb, ss)
                    drain(ss, rowsb)

                    @pl.when(g + 2 < groups)
                    def _():
                        load_sd(g + 2, sdb)
                        fire_gathers(slab_ref, sdb, rowsb, sg)
            plsc.subcore_barrier()

            pltpu.sync_copy(acc.at[pl.ds(zrow, npt), :],
                            out_hbm.at[slab * _NC + c, pl.ds(zrow, npt), :])

    out_type = jax.ShapeDtypeStruct((nslab * _NC, _NPAD, _LANES),
                                    jnp.float32)
    return pl.kernel(body, out_type=out_type, mesh=mesh,
                     scratch_types=scratch,
                     compiler_params=pltpu.CompilerParams(
                         use_tc_tiling_on_sc=False))


_R = 512


@functools.lru_cache(maxsize=None)
def _make_pre(nslab_in, d_out, n_nodes):
    dp = 8 * d_out
    npk = _NPAD // 8
    nblk = npk // _R
    nreal = n_nodes // 8

    def body(*refs):
        agg_ref = refs[0]
        hs = refs[1:1 + nslab_in]
        wrel, wroot, bias = refs[1 + nslab_in:4 + nslab_in]
        pre_ref, stats_ref, acc = refs[4 + nslab_in:]
        i = pl.program_id(0)
        o = bias[...]
        for s in range(nslab_in):
            aggs = (agg_ref[2 * s] + agg_ref[2 * s + 1]).astype(jnp.bfloat16)
            o = o + jnp.dot(aggs, wrel[s].astype(jnp.bfloat16),
                            preferred_element_type=jnp.float32)
            o = o + jnp.dot(hs[s][...].astype(jnp.bfloat16),
                            wroot[s].astype(jnp.bfloat16),
                            preferred_element_type=jnp.float32)
        pre_ref[...] = o
        rid = lax.broadcasted_iota(jnp.int32, (_R, 1), 0) + i * _R
        valid = (rid < nreal).astype(jnp.float32)
        ov = o * valid
        s1f = jnp.sum(ov, axis=0, keepdims=True)
        s2f = jnp.sum(ov * o, axis=0, keepdims=True)
        s1 = s1f[:, 0:d_out]
        s2 = s2f[:, 0:d_out]
        for k in range(1, 8):
            s1 = s1 + s1f[:, k * d_out:(k + 1) * d_out]
            s2 = s2 + s2f[:, k * d_out:(k + 1) * d_out]
        st = jnp.concatenate([s1, s2], axis=0)

        @pl.when(i == 0)
        def _():
            acc[...] = st

        @pl.when(i > 0)
        def _():
            acc[...] += st

        @pl.when(i == nblk - 1)
        def _():
            stats_ref[...] = acc[...]

    in_specs = (
        [pl.BlockSpec((nslab_in * 2, _R, 8 * _LANES), lambda i: (0, i, 0))]
        + [pl.BlockSpec((_R, 8 * _LANES), lambda i: (i, 0))] * nslab_in
        + [pl.BlockSpec((nslab_in, 8 * _LANES, dp), lambda i: (0, 0, 0)),
           pl.BlockSpec((nslab_in, 8 * _LANES, dp), lambda i: (0, 0, 0)),
           pl.BlockSpec((1, dp), lambda i: (0, 0))]
    )
    out_specs = [pl.BlockSpec((_R, dp), lambda i: (i, 0)),
                 pl.BlockSpec((2, d_out), lambda i: (0, 0))]
    out_shape = [jax.ShapeDtypeStruct((npk, dp), jnp.float32),
                 jax.ShapeDtypeStruct((2, d_out), jnp.float32)]

    return pl.pallas_call(
        body, grid=(nblk,), in_specs=in_specs, out_specs=out_specs,
        out_shape=out_shape,
        scratch_shapes=[pltpu.VMEM((2, d_out), jnp.float32)])


@functools.lru_cache(maxsize=None)
def _make_norm(d, n_nodes):
    dp = 8 * d
    nslab_out = d // _LANES
    npk = _NPAD // 8
    nblk = npk // _R

    def body(pre_ref, stats_ref, bng, bnb, gng, gnb, gna, *outs):
        n = jnp.float32(n_nodes)
        m = stats_ref[0:1, :] / n
        v = stats_ref[1:2, :] / n - m * m
        bg = bng[...]
        bb = bnb[...]
        gg = gng[...]
        gb = gnb[...]
        ga = gna[...]
        inv_s = lax.rsqrt(v + _EPS)
        cc = bb * (1.0 - ga)
        v2 = bg * bg * v * inv_s * inv_s + cc * cc
        t_inv = lax.rsqrt(v2 + _EPS)
        a_coef = gg * bg * inv_s * t_inv
        b_coef = gg * cc * t_inv + gb - a_coef * m
        a_t = jnp.tile(a_coef, (1, 8))
        b_t = jnp.tile(b_coef, (1, 8))
        h = jnp.maximum(pre_ref[...] * a_t + b_t, 0.0)
        for s in range(nslab_out):
            if nslab_out == 1:
                outs[s][...] = h
            else:
                outs[s][...] = jnp.concatenate(
                    [h[:, k * d + s * _LANES:k * d + (s + 1) * _LANES]
                     for k in range(8)], axis=1)

    in_specs = (
        [pl.BlockSpec((_R, dp), lambda i: (i, 0)),
         pl.BlockSpec((2, d), lambda i: (0, 0))]
        + [pl.BlockSpec((1, d), lambda i: (0, 0))] * 5
    )
    out_specs = [pl.BlockSpec((_R, 8 * _LANES), lambda i: (i, 0))] * nslab_out
    out_shape = [jax.ShapeDtypeStruct((npk, 8 * _LANES), jnp.float32)] * nslab_out

    return pl.pallas_call(
        body, grid=(nblk,), in_specs=in_specs, out_specs=out_specs,
        out_shape=out_shape)


@functools.lru_cache(maxsize=None)
def _make_heads(nslab_in, d_out, n_nodes):
    dp = 8 * d_out
    npk = _NPAD // 8
    nblk = npk // _R

    def body(*refs):
        agg_ref = refs[0]
        hs = refs[1:1 + nslab_in]
        wrel_mu, wroot_mu, b_mu, wrel_lv, wroot_lv, b_lv = \
            refs[1 + nslab_in:7 + nslab_in]
        mu_ref, lv_ref = refs[7 + nslab_in:]
        mu = b_mu[...]
        lv = b_lv[...]
        for s in range(nslab_in):
            aggs = (agg_ref[2 * s] + agg_ref[2 * s + 1]).astype(jnp.bfloat16)
            hss = hs[s][...].astype(jnp.bfloat16)
            mu = mu + jnp.dot(aggs, wrel_mu[s].astype(jnp.bfloat16),
                              preferred_element_type=jnp.float32)
            mu = mu + jnp.dot(hss, wroot_mu[s].astype(jnp.bfloat16),
                              preferred_element_type=jnp.float32)
            lv = lv + jnp.dot(aggs, wrel_lv[s].astype(jnp.bfloat16),
                              preferred_element_type=jnp.float32)
            lv = lv + jnp.dot(hss, wroot_lv[s].astype(jnp.bfloat16),
                              preferred_element_type=jnp.float32)
        mu_ref[...] = mu
        lv_ref[...] = lv

    wspec = pl.BlockSpec((nslab_in, 8 * _LANES, dp), lambda i: (0, 0, 0))
    bspec = pl.BlockSpec((1, dp), lambda i: (0, 0))
    in_specs = (
        [pl.BlockSpec((nslab_in * 2, _R, 8 * _LANES), lambda i: (0, i, 0))]
        + [pl.BlockSpec((_R, 8 * _LANES), lambda i: (i, 0))] * nslab_in
        + [wspec, wspec, bspec, wspec, wspec, bspec]
    )
    out_specs = [pl.BlockSpec((_R, dp), lambda i: (i, 0))] * 2
    out_shape = [jax.ShapeDtypeStruct((n_nodes // 8, dp), jnp.float32)] * 2

    return pl.pallas_call(
        body, grid=(nblk,), in_specs=in_specs, out_specs=out_specs,
        out_shape=out_shape)


@functools.lru_cache(maxsize=None)
def _make_widen(n_nodes):
    npk = _NPAD // 8
    nblk = npk // _R

    def body(x_ref, out_ref):
        xb = x_ref[...]
        z = jnp.zeros((_R, _LANES - 1), jnp.float32)
        cols = []
        for i in range(8):
            cols.append(xb[:, i:i + 1])
            cols.append(z)
        out_ref[...] = jnp.concatenate(cols, axis=1)

    return pl.pallas_call(
        body, grid=(nblk,),
        in_specs=[pl.BlockSpec((_R, 8), lambda i: (i, 0))],
        out_specs=[pl.BlockSpec((_R, 8 * _LANES), lambda i: (i, 0))],
        out_shape=[jax.ShapeDtypeStruct((npk, 8 * _LANES), jnp.float32)])


def _wpack(w, nslab):
    eye = jnp.eye(8, dtype=w.dtype)
    return jnp.stack([jnp.kron(eye, w[16 * s:16 * (s + 1), :])
                      for s in range(nslab)])


def kernel(x, edge_index,
           conv1_w_rel, conv1_w_root, conv1_b,
           conv2_w_rel, conv2_w_root, conv2_b,
           conv3_w_rel, conv3_w_root, conv3_b,
           conv_mu_w_rel, conv_mu_w_root, conv_mu_b,
           conv_logvar_w_rel, conv_logvar_w_root, conv_logvar_b,
           bn1_g, bn1_b, bn2_g, bn2_b, bn3_g, bn3_b,
           gn1_g, gn1_b, gn1_a, gn2_g, gn2_b, gn2_a,
           gn3_g, gn3_b, gn3_a):
    n = x.shape[0]
    e = edge_index.shape[1]
    n_rows = e // _CHUNK
    sd = edge_index.reshape(2, n_rows, _CHUNK)

    row2 = lambda a: a.reshape(1, -1)
    tile8 = lambda a: jnp.tile(a, 8).reshape(1, -1)
    unpk = lambda a: a.reshape(_NPAD, _LANES)
    pk = lambda a: a.reshape(a.shape[0], _NPAD // 8, 8 * _LANES)

    x8 = jnp.pad(x.reshape(n // 8, 8), ((0, (_NPAD - n) // 8), (0, 0)))
    (x16p,) = _make_widen(n)(x8)
    x16 = unpk(x16p)
    w1r = jnp.pad(conv1_w_rel, ((0, _LANES - 1), (0, 0)))
    w1t = jnp.pad(conv1_w_root, ((0, _LANES - 1), (0, 0)))
    agg1 = pk(_make_sc_agg(1, n, n_rows)(x16, sd))
    pre1, st1 = _make_pre(1, 16, n)(agg1, x16p, _wpack(w1r, 1),
                                    _wpack(w1t, 1), tile8(conv1_b))
    (h1,) = _make_norm(16, n)(pre1, st1, row2(bn1_g), row2(bn1_b),
                              row2(gn1_g), row2(gn1_b), row2(gn1_a))

    agg2 = pk(_make_sc_agg(1, n, n_rows)(unpk(h1), sd))
    pre2, st2 = _make_pre(1, 32, n)(agg2, h1, _wpack(conv2_w_rel, 1),
                                    _wpack(conv2_w_root, 1), tile8(conv2_b))
    h2a, h2b = _make_norm(32, n)(pre2, st2, row2(bn2_g), row2(bn2_b),
                                 row2(gn2_g), row2(gn2_b), row2(gn2_a))

    agg3 = pk(_make_sc_agg(2, n, n_rows)(unpk(h2a), unpk(h2b), sd))
    pre3, st3 = _make_pre(2, 64, n)(agg3, h2a, h2b, _wpack(conv3_w_rel, 2),
                                    _wpack(conv3_w_root, 2), tile8(conv3_b))
    h3 = _make_norm(64, n)(pre3, st3, row2(bn3_g), row2(bn3_b),
                           row2(gn3_g), row2(gn3_b), row2(gn3_a))

    agg4 = pk(_make_sc_agg(4, n, n_rows)(unpk(h3[0]), unpk(h3[1]),
                                         unpk(h3[2]), unpk(h3[3]), sd))
    mup, lvp = _make_heads(4, 64, n)(
        agg4, h3[0], h3[1], h3[2], h3[3],
        _wpack(conv_mu_w_rel, 4), _wpack(conv_mu_w_root, 4),
        tile8(conv_mu_b),
        _wpack(conv_logvar_w_rel, 4), _wpack(conv_logvar_w_root, 4),
        tile8(conv_logvar_b))
    return mup.reshape(n, 64), lvp.reshape(n, 64)


# speedup vs baseline: 2.6666x; 1.0146x over previous
import functools

import jax
import jax.numpy as jnp
from jax import lax
from jax.experimental import pallas as pl
from jax.experimental.pallas import tpu as pltpu
from jax.experimental.pallas import tpu_sc as plsc

_LANES = 16
_CHUNK = 128
_GROUP = 5
_NC = 2
_NS = 16
_NW = _NC * _NS
_NPAD = 102400
_ZROWS = 160
_EPS = 1e-5


@functools.lru_cache(maxsize=None)
def _make_sc_agg(nslab, n_nodes, n_rows):
    rows_per_tile = n_rows // _NW
    groups = rows_per_tile // _GROUP
    npt = _NPAD // _NS
    nzcopy = npt // _ZROWS

    mesh = plsc.VectorSubcoreMesh(
        core_axis_name="c", subcore_axis_name="s",
        num_cores=_NC, num_subcores=_NS)

    gb = _GROUP * _CHUNK

    scratch = [
        pltpu.VMEM((2, _GROUP, _CHUNK), jnp.int32),
        pltpu.VMEM((2, _GROUP, _CHUNK), jnp.int32),
        pltpu.VMEM((gb, _LANES), jnp.float32),
        pltpu.VMEM((gb, _LANES), jnp.float32),
        pltpu.VMEM((_ZROWS, _LANES), jnp.float32),
        pltpu.VMEM_SHARED((_NPAD, _LANES), jnp.float32),
        pltpu.SemaphoreType.DMA,
        pltpu.SemaphoreType.DMA,
        pltpu.SemaphoreType.DMA,
        pltpu.SemaphoreType.DMA,
    ]

    def body(*refs):
        slabs = refs[:nslab]
        sd_hbm, out_hbm = refs[nslab:nslab + 2]
        sdb0, sdb1, rows0, rows1, zbuf, acc, sg0, sg1, ss0, ss1 = \
            refs[nslab + 2:]
        c = lax.axis_index("c")
        s = lax.axis_index("s")
        wid = c * _NS + s
        rowbase = wid * rows_per_tile
        zrow = s * npt
        bufs = ((sdb0, rows0, sg0, ss0), (sdb1, rows1, sg1, ss1))

        def load_sd(g, sdb):
            pltpu.sync_copy(sd_hbm.at[:, pl.ds(rowbase + g * _GROUP, _GROUP), :],
                            sdb)

        def fire_gathers(slab_ref, sdb, rowsb, sem):
            for j in range(_GROUP):
                pltpu.async_copy(slab_ref.at[sdb.at[0, j]],
                                 rowsb.at[pl.ds(j * _CHUNK, _CHUNK), :], sem)

        def fire_scatters(sdb, rowsb, sem):
            for j in range(_GROUP):
                pltpu.async_copy(rowsb.at[pl.ds(j * _CHUNK, _CHUNK), :],
                                 acc.at[sdb.at[1, j]], sem, add=True)

        def drain(sem, rowsb):
            pltpu.make_async_copy(out_hbm.at[0, pl.ds(0, gb), :], rowsb,
                                  sem).wait()

        @pl.loop(0, _ZROWS)
        def _(i):
            zbuf[i, :] = jnp.zeros((_LANES,), jnp.float32)

        for slab in range(nslab):
            for z in range(nzcopy):
                pltpu.sync_copy(zbuf, acc.at[pl.ds(zrow + z * _ZROWS, _ZROWS), :])
            plsc.subcore_barrier()

            slab_ref = slabs[slab]
            for b in range(2):
                sdb, rowsb, sg, _ = bufs[b]
                load_sd(b, sdb)
                fire_gathers(slab_ref, sdb, rowsb, sg)

            @pl.loop(0, groups, step=2)
            def _(gg):
                for b in range(2):
                    sdb, rowsb, sg, ss = bufs[b]
                    g = gg + b
                    drain(sg, rowsb)
                    fire_scatters(sdb, rowsb, ss)
                    drain(ss, rowsb)

                    @pl.when(g + 2 < groups)
                    def _():
                        load_sd(g + 2, sdb)
                        fire_gathers(slab_ref, sdb, rowsb, sg)
            plsc.subcore_barrier()

            pltpu.sync_copy(acc.at[pl.ds(zrow, npt), :],
                            out_hbm.at[slab * _NC + c, pl.ds(zrow, npt), :])

    out_type = jax.ShapeDtypeStruct((nslab * _NC, _NPAD, _LANES),
                                    jnp.float32)
    return pl.kernel(body, out_type=out_type, mesh=mesh,
                     scratch_types=scratch,
                     compiler_params=pltpu.CompilerParams(
                         use_tc_tiling_on_sc=False))


_R = 512


@functools.lru_cache(maxsize=None)
def _make_pre(nslab_in, d_out, n_nodes):
    dp = 8 * d_out
    npk = _NPAD // 8
    nblk = npk // _R
    nreal = n_nodes // 8

    def body(*refs):
        agg_ref = refs[0]
        hs = refs[1:1 + nslab_in]
        wrel, wroot, bias = refs[1 + nslab_in:4 + nslab_in]
        pre_ref, stats_ref, acc = refs[4 + nslab_in:]
        i = pl.program_id(0)
        o = bias[...]
        for s in range(nslab_in):
            aggs = (agg_ref[2 * s] + agg_ref[2 * s + 1]).astype(jnp.bfloat16)
            o = o + jnp.dot(aggs, wrel[s].astype(jnp.bfloat16),
                            preferred_element_type=jnp.float32)
            o = o + jnp.dot(hs[s][...].astype(jnp.bfloat16),
                            wroot[s].astype(jnp.bfloat16),
                            preferred_element_type=jnp.float32)
        pre_ref[...] = o
        rid = lax.broadcasted_iota(jnp.int32, (_R, 1), 0) + i * _R
        valid = (rid < nreal).astype(jnp.float32)
        ov = o * valid
        s1f = jnp.sum(ov, axis=0, keepdims=True)
        s2f = jnp.sum(ov * o, axis=0, keepdims=True)
        s1 = s1f[:, 0:d_out]
        s2 = s2f[:, 0:d_out]
        for k in range(1, 8):
            s1 = s1 + s1f[:, k * d_out:(k + 1) * d_out]
            s2 = s2 + s2f[:, k * d_out:(k + 1) * d_out]
        st = jnp.concatenate([s1, s2], axis=0)

        @pl.when(i == 0)
        def _():
            acc[...] = st

        @pl.when(i > 0)
        def _():
            acc[...] += st

        @pl.when(i == nblk - 1)
        def _():
            stats_ref[...] = acc[...]

    in_specs = (
        [pl.BlockSpec((nslab_in * 2, _R, 8 * _LANES), lambda i: (0, i, 0))]
        + [pl.BlockSpec((_R, 8 * _LANES), lambda i: (i, 0))] * nslab_in
        + [pl.BlockSpec((nslab_in, 8 * _LANES, dp), lambda i: (0, 0, 0)),
           pl.BlockSpec((nslab_in, 8 * _LANES, dp), lambda i: (0, 0, 0)),
           pl.BlockSpec((1, dp), lambda i: (0, 0))]
    )
    out_specs = [pl.BlockSpec((_R, dp), lambda i: (i, 0)),
                 pl.BlockSpec((2, d_out), lambda i: (0, 0))]
    out_shape = [jax.ShapeDtypeStruct((npk, dp), jnp.float32),
                 jax.ShapeDtypeStruct((2, d_out), jnp.float32)]

    return pl.pallas_call(
        body, grid=(nblk,), in_specs=in_specs, out_specs=out_specs,
        out_shape=out_shape,
        scratch_shapes=[pltpu.VMEM((2, d_out), jnp.float32)])


@functools.lru_cache(maxsize=None)
def _make_norm(d, n_nodes):
    dp = 8 * d
    nslab_out = d // _LANES
    npk = _NPAD // 8
    nblk = npk // _R

    def body(pre_ref, stats_ref, bng, bnb, gng, gnb, gna, *outs):
        n = jnp.float32(n_nodes)
        m = stats_ref[0:1, :] / n
        v = stats_ref[1:2, :] / n - m * m
        bg = bng[...]
        bb = bnb[...]
        gg = gng[...]
        gb = gnb[...]
        ga = gna[...]
        inv_s = lax.rsqrt(v + _EPS)
        cc = bb * (1.0 - ga)
        v2 = bg * bg * v * inv_s * inv_s + cc * cc
        t_inv = lax.rsqrt(v2 + _EPS)
        a_coef = gg * bg * inv_s * t_inv
        b_coef = gg * cc * t_inv + gb - a_coef * m
        a_t = jnp.tile(a_coef, (1, 8))
        b_t = jnp.tile(b_coef, (1, 8))
        h = jnp.maximum(pre_ref[...] * a_t + b_t, 0.0)
        for s in range(nslab_out):
            if nslab_out == 1:
                outs[s][...] = h
            else:
                outs[s][...] = jnp.concatenate(
                    [h[:, k * d + s * _LANES:k * d + (s + 1) * _LANES]
                     for k in range(8)], axis=1)

    in_specs = (
        [pl.BlockSpec((_R, dp), lambda i: (i, 0)),
         pl.BlockSpec((2, d), lambda i: (0, 0))]
        + [pl.BlockSpec((1, d), lambda i: (0, 0))] * 5
    )
    out_specs = [pl.BlockSpec((_R, 8 * _LANES), lambda i: (i, 0))] * nslab_out
    out_shape = [jax.ShapeDtypeStruct((npk, 8 * _LANES), jnp.float32)] * nslab_out

    return pl.pallas_call(
        body, grid=(nblk,), in_specs=in_specs, out_specs=out_specs,
        out_shape=out_shape)


@functools.lru_cache(maxsize=None)
def _make_heads(nslab_in, d_out, n_nodes):
    dp = 8 * d_out
    npk = _NPAD // 8
    nblk = npk // _R

    def body(*refs):
        agg_ref = refs[0]
        hs = refs[1:1 + nslab_in]
        wrel_mu, wroot_mu, b_mu, wrel_lv, wroot_lv, b_lv = \
            refs[1 + nslab_in:7 + nslab_in]
        mu_ref, lv_ref = refs[7 + nslab_in:]
        mu = b_mu[...]
        lv = b_lv[...]
        for s in range(nslab_in):
            aggs = (agg_ref[2 * s] + agg_ref[2 * s + 1]).astype(jnp.bfloat16)
            hss = hs[s][...].astype(jnp.bfloat16)
            mu = mu + jnp.dot(aggs, wrel_mu[s].astype(jnp.bfloat16),
                              preferred_element_type=jnp.float32)
            mu = mu + jnp.dot(hss, wroot_mu[s].astype(jnp.bfloat16),
                              preferred_element_type=jnp.float32)
            lv = lv + jnp.dot(aggs, wrel_lv[s].astype(jnp.bfloat16),
                              preferred_element_type=jnp.float32)
            lv = lv + jnp.dot(hss, wroot_lv[s].astype(jnp.bfloat16),
                              preferred_element_type=jnp.float32)
        mu_ref[...] = mu
        lv_ref[...] = lv

    wspec = pl.BlockSpec((nslab_in, 8 * _LANES, dp), lambda i: (0, 0, 0))
    bspec = pl.BlockSpec((1, dp), lambda i: (0, 0))
    in_specs = (
        [pl.BlockSpec((nslab_in * 2, _R, 8 * _LANES), lambda i: (0, i, 0))]
        + [pl.BlockSpec((_R, 8 * _LANES), lambda i: (i, 0))] * nslab_in
        + [wspec, wspec, bspec, wspec, wspec, bspec]
    )
    out_specs = [pl.BlockSpec((_R, dp), lambda i: (i, 0))] * 2
    out_shape = [jax.ShapeDtypeStruct((n_nodes // 8, dp), jnp.float32)] * 2

    return pl.pallas_call(
        body, grid=(nblk,), in_specs=in_specs, out_specs=out_specs,
        out_shape=out_shape)


@functools.lru_cache(maxsize=None)
def _make_widen(n_nodes):
    npk = _NPAD // 8
    nblk = npk // _R

    def body(x_ref, out_ref):
        xb = x_ref[...]
        z = jnp.zeros((_R, _LANES - 1), jnp.float32)
        cols = []
        for i in range(8):
            cols.append(xb[:, i:i + 1])
            cols.append(z)
        out_ref[...] = jnp.concatenate(cols, axis=1)

    return pl.pallas_call(
        body, grid=(nblk,),
        in_specs=[pl.BlockSpec((_R, 8), lambda i: (i, 0))],
        out_specs=[pl.BlockSpec((_R, 8 * _LANES), lambda i: (i, 0))],
        out_shape=[jax.ShapeDtypeStruct((npk, 8 * _LANES), jnp.float32)])


def _wpack(w, nslab):
    eye = jnp.eye(8, dtype=w.dtype)
    return jnp.stack([jnp.kron(eye, w[16 * s:16 * (s + 1), :])
                      for s in range(nslab)])


def kernel(x, edge_index,
           conv1_w_rel, conv1_w_root, conv1_b,
           conv2_w_rel, conv2_w_root, conv2_b,
           conv3_w_rel, conv3_w_root, conv3_b,
           conv_mu_w_rel, conv_mu_w_root, conv_mu_b,
           conv_logvar_w_rel, conv_logvar_w_root, conv_logvar_b,
           bn1_g, bn1_b, bn2_g, bn2_b, bn3_g, bn3_b,
           gn1_g, gn1_b, gn1_a, gn2_g, gn2_b, gn2_a,
           gn3_g, gn3_b, gn3_a):
    n = x.shape[0]
    e = edge_index.shape[1]
    e_pad = _NW * 400 * _CHUNK
    ar = jnp.arange(e_pad - e, dtype=jnp.int32)
    fill_src = jax.lax.rem(ar * 4099, jnp.int32(n))
    fill_dst = n + jax.lax.rem(ar, jnp.int32(_NPAD - n))
    fill = jnp.stack([fill_src, fill_dst])
    n_rows = e_pad // _CHUNK
    sd = jnp.concatenate([edge_index, fill], axis=1).reshape(
        2, n_rows, _CHUNK)

    row2 = lambda a: a.reshape(1, -1)
    tile8 = lambda a: jnp.tile(a, 8).reshape(1, -1)
    unpk = lambda a: a.reshape(_NPAD, _LANES)
    pk = lambda a: a.reshape(a.shape[0], _NPAD // 8, 8 * _LANES)

    x8 = jnp.pad(x.reshape(n // 8, 8), ((0, (_NPAD - n) // 8), (0, 0)))
    (x16p,) = _make_widen(n)(x8)
    x16 = unpk(x16p)
    w1r = jnp.pad(conv1_w_rel, ((0, _LANES - 1), (0, 0)))
    w1t = jnp.pad(conv1_w_root, ((0, _LANES - 1), (0, 0)))
    agg1 = pk(_make_sc_agg(1, n, n_rows)(x16, sd))
    pre1, st1 = _make_pre(1, 16, n)(agg1, x16p, _wpack(w1r, 1),
                                    _wpack(w1t, 1), tile8(conv1_b))
    (h1,) = _make_norm(16, n)(pre1, st1, row2(bn1_g), row2(bn1_b),
                              row2(gn1_g), row2(gn1_b), row2(gn1_a))

    agg2 = pk(_make_sc_agg(1, n, n_rows)(unpk(h1), sd))
    pre2, st2 = _make_pre(1, 32, n)(agg2, h1, _wpack(conv2_w_rel, 1),
                                    _wpack(conv2_w_root, 1), tile8(conv2_b))
    h2a, h2b = _make_norm(32, n)(pre2, st2, row2(bn2_g), row2(bn2_b),
                                 row2(gn2_g), row2(gn2_b), row2(gn2_a))

    agg3 = pk(_make_sc_agg(2, n, n_rows)(unpk(h2a), unpk(h2b), sd))
    pre3, st3 = _make_pre(2, 64, n)(agg3, h2a, h2b, _wpack(conv3_w_rel, 2),
                                    _wpack(conv3_w_root, 2), tile8(conv3_b))
    h3 = _make_norm(64, n)(pre3, st3, row2(bn3_g), row2(bn3_b),
                           row2(gn3_g), row2(gn3_b), row2(gn3_a))

    agg4 = pk(_make_sc_agg(4, n, n_rows)(unpk(h3[0]), unpk(h3[1]),
                                         unpk(h3[2]), unpk(h3[3]), sd))
    mup, lvp = _make_heads(4, 64, n)(
        agg4, h3[0], h3[1], h3[2], h3[3],
        _wpack(conv_mu_w_rel, 4), _wpack(conv_mu_w_root, 4),
        tile8(conv_mu_b),
        _wpack(conv_logvar_w_rel, 4), _wpack(conv_logvar_w_root, 4),
        tile8(conv_logvar_b))
    return mup.reshape(n, 64), lvp.reshape(n, 64)
